# baseline (speedup 1.0000x reference)
import functools

import jax
import jax.numpy as jnp
import numpy as np
from jax import lax
from jax.experimental import pallas as pl
from jax.experimental.pallas import tpu as pltpu

D = 1024
H = 16
E = 8
FF = 2048
S = 2048
DK = D // H
NCLS = 10


def _bdot(a, b):
    return jnp.dot(a, b, preferred_element_type=jnp.float32,
                   precision=lax.Precision.DEFAULT)


BLK = 128
P = 2 * S + E * BLK
NBLK = P // BLK


def _sc_gather_rows(table, idx, out_rows, ch):
    from jax.experimental.pallas import tpu_sc as plsc

    info = plsc.get_sparse_core_info()
    nc_, ns_ = info.num_cores, info.num_subcores
    nw = nc_ * ns_
    b_per_w = out_rows // nw
    nch = b_per_w // ch
    d = table.shape[1]
    mesh = plsc.VectorSubcoreMesh(core_axis_name="c", subcore_axis_name="s")

    nbuf = min(2, nch)

    @functools.partial(
        pl.kernel,
        mesh=mesh,
        out_type=jax.ShapeDtypeStruct((out_rows, d), jnp.float32),
        scratch_types=[
            pltpu.VMEM((b_per_w,), jnp.int32),
            pltpu.VMEM((nbuf, ch, d), jnp.float32),
            pltpu.SemaphoreType.DMA,
            pltpu.SemaphoreType.DMA,
        ],
    )
    def k(table_hbm, idx_hbm, out_hbm, idx_v, rows_v, sem0, sem1):
        wid = lax.axis_index("s") * nc_ + lax.axis_index("c")
        base = wid * b_per_w
        sems = [sem0, sem1]
        pltpu.sync_copy(idx_hbm.at[pl.ds(base, b_per_w)], idx_v)
        handles = {}
        for c in range(nbuf):
            handles[c] = pltpu.async_copy(
                table_hbm.at[idx_v.at[pl.ds(c * ch, ch)]],
                rows_v.at[c % nbuf], sems[c % nbuf])
        for c in range(nch):
            handles[c].wait()
            pltpu.sync_copy(rows_v.at[c % nbuf],
                            out_hbm.at[pl.ds(base + c * ch, ch)])
            nxt = c + nbuf
            if nxt < nch:
                handles[nxt] = pltpu.async_copy(
                    table_hbm.at[idx_v.at[pl.ds(nxt * ch, ch)]],
                    rows_v.at[nxt % nbuf], sems[nxt % nbuf])

    return k(table, idx)


def _rmsnorm_kernel(x_ref, w_ref, o_ref):
    x = x_ref[...]
    r = lax.rsqrt(jnp.mean(x * x, axis=-1, keepdims=True) + 1e-6)
    o_ref[...] = x * r * w_ref[...]


def _rmsnorm(x, w, bm=256, interpret=False):
    m, d = x.shape
    return pl.pallas_call(
        _rmsnorm_kernel,
        grid=(m // bm,),
        in_specs=[
            pl.BlockSpec((bm, d), lambda i: (i, 0)),
            pl.BlockSpec((1, d), lambda i: (0, 0)),
        ],
        out_specs=pl.BlockSpec((bm, d), lambda i: (i, 0)),
        out_shape=jax.ShapeDtypeStruct((m, d), jnp.float32),
        interpret=interpret,
    )(x, w.reshape(1, d))


def _mm_kernel(x_ref, w_ref, o_ref):
    o_ref[...] = _bdot(x_ref[...], w_ref[...])


def _matmul(x, w, bm=512, interpret=False):
    m, k = x.shape
    _, n = w.shape
    return pl.pallas_call(
        _mm_kernel,
        grid=(m // bm,),
        in_specs=[
            pl.BlockSpec((bm, k), lambda i: (i, 0)),
            pl.BlockSpec((k, n), lambda i: (0, 0)),
        ],
        out_specs=pl.BlockSpec((bm, n), lambda i: (i, 0)),
        out_shape=jax.ShapeDtypeStruct((m, n), jnp.float32),
        interpret=interpret,
    )(x, w)


def _mm_add_kernel(x_ref, w_ref, r_ref, o_ref):
    o_ref[...] = r_ref[...] + _bdot(x_ref[...], w_ref[...])


def _matmul_add(x, w, res, bm=512, interpret=False):
    m, k = x.shape
    _, n = w.shape
    return pl.pallas_call(
        _mm_add_kernel,
        grid=(m // bm,),
        in_specs=[
            pl.BlockSpec((bm, k), lambda i: (i, 0)),
            pl.BlockSpec((k, n), lambda i: (0, 0)),
            pl.BlockSpec((bm, n), lambda i: (i, 0)),
        ],
        out_specs=pl.BlockSpec((bm, n), lambda i: (i, 0)),
        out_shape=jax.ShapeDtypeStruct((m, n), jnp.float32),
        interpret=interpret,
    )(x, w, res)


def _attn_kernel(q_ref, k_ref, v_ref, o_ref):
    q = q_ref[0]
    k = k_ref[0]
    v = v_ref[0]
    s = _bdot(q, k.T) * (1.0 / np.sqrt(DK))
    m = jnp.max(s, axis=-1, keepdims=True)
    e = jnp.exp(s - m)
    p = e / jnp.sum(e, axis=-1, keepdims=True)
    o_ref[0] = _bdot(p, v)


def _attention(q, k, v, bq=512, interpret=False):
    return pl.pallas_call(
        _attn_kernel,
        grid=(H, S // bq),
        in_specs=[
            pl.BlockSpec((1, bq, DK), lambda h, i: (h, i, 0)),
            pl.BlockSpec((1, S, DK), lambda h, i: (h, 0, 0)),
            pl.BlockSpec((1, S, DK), lambda h, i: (h, 0, 0)),
        ],
        out_specs=pl.BlockSpec((1, bq, DK), lambda h, i: (h, i, 0)),
        out_shape=jax.ShapeDtypeStruct((H, S, DK), jnp.float32),
        interpret=interpret,
    )(q, k, v)


def _router_kernel(x_ref, w_ref, o_ref):
    logits = jnp.dot(x_ref[...], w_ref[...], preferred_element_type=jnp.float32)
    lane = lax.broadcasted_iota(jnp.int32, logits.shape, 1)
    neg = jnp.float32(-1e30)
    logits = jnp.where(lane < E, logits, neg)
    m = jnp.max(logits, axis=-1, keepdims=True)
    ex = jnp.where(lane < E, jnp.exp(logits - m), 0.0)
    probs = ex / jnp.sum(ex, axis=-1, keepdims=True)
    i1 = jnp.argmax(probs, axis=-1)
    m1 = jnp.max(probs, axis=-1)
    p2 = jnp.where(lane == i1[:, None], neg, jnp.where(lane < E, probs, neg))
    i2 = jnp.argmax(p2, axis=-1)
    m2 = jnp.max(p2, axis=-1)
    e1 = jnp.exp(m1)
    e2 = jnp.exp(m2)
    w1 = e1 / (e1 + e2)
    w2 = e2 / (e1 + e2)
    tw = w1[:, None] * (lane == i1[:, None]) + w2[:, None] * (lane == i2[:, None])
    o_ref[...] = tw.astype(jnp.float32)


def _router(x, router_w, bm=256, interpret=False):
    m, d = x.shape
    return pl.pallas_call(
        _router_kernel,
        grid=(m // bm,),
        in_specs=[
            pl.BlockSpec((bm, d), lambda i: (i, 0)),
            pl.BlockSpec((d, 128), lambda i: (0, 0)),
        ],
        out_specs=pl.BlockSpec((bm, 128), lambda i: (i, 0)),
        out_shape=jax.ShapeDtypeStruct((m, 128), jnp.float32),
        interpret=interpret,
    )(x, router_w)


def _swiglu_kernel(x_ref, w1_ref, w3_ref, w2_ref, o_ref):
    x = x_ref[...]
    a = _bdot(x, w1_ref[...])
    b = _bdot(x, w3_ref[...])
    h = (a * jax.lax.logistic(a)) * b
    o_ref[...] = _bdot(h, w2_ref[...])


def _swiglu(x, w1, w3, w2, bm=512, interpret=False):
    m, d = x.shape
    ff = w1.shape[1]
    return pl.pallas_call(
        _swiglu_kernel,
        grid=(m // bm,),
        in_specs=[
            pl.BlockSpec((bm, d), lambda i: (i, 0)),
            pl.BlockSpec((d, ff), lambda i: (0, 0)),
            pl.BlockSpec((d, ff), lambda i: (0, 0)),
            pl.BlockSpec((ff, d), lambda i: (0, 0)),
        ],
        out_specs=pl.BlockSpec((bm, d), lambda i: (i, 0)),
        out_shape=jax.ShapeDtypeStruct((m, d), jnp.float32),
        interpret=interpret,
    )(x, w1, w3, w2)


_HI = lax.Precision.HIGHEST


def _dispatch_kernel(h_ref, r_ref, slot_ref, w1b_ref, w2b_ref, eblk_ref, c_scr):
    h = h_ref[...]
    logits = jnp.dot(h, r_ref[...], preferred_element_type=jnp.float32)
    lane = lax.broadcasted_iota(jnp.int32, logits.shape, 1)
    neg = jnp.float32(-1e30)
    logits = jnp.where(lane < E, logits, neg)
    m = jnp.max(logits, axis=-1, keepdims=True)
    ex = jnp.where(lane < E, jnp.exp(logits - m), 0.0)
    probs = ex / jnp.sum(ex, axis=-1, keepdims=True)
    pm = jnp.where(lane < E, probs, neg)
    i1 = jnp.argmax(pm, axis=-1)
    m1 = jnp.max(pm, axis=-1)
    p2 = jnp.where(lane == i1[:, None], neg, pm)
    i2 = jnp.argmax(p2, axis=-1)
    m2 = jnp.max(p2, axis=-1)
    e1 = jnp.exp(m1)
    e2 = jnp.exp(m2)
    wa = e1 / (e1 + e2)
    wb = e2 / (e1 + e2)
    w1b_ref[...] = jnp.broadcast_to(wa[:, None], w1b_ref.shape)
    w2b_ref[...] = jnp.broadcast_to(wb[:, None], w2b_ref.shape)

    icat = jnp.concatenate([i1[:, None], i2[:, None]], axis=0)
    a_tot = 2 * S
    ch = 512
    lane128 = lax.broadcasted_iota(jnp.int32, (1, 128), 1)
    r0 = lax.broadcasted_iota(jnp.int32, (ch, ch), 0)
    c0 = lax.broadcasted_iota(jnp.int32, (ch, ch), 1)
    tri = (r0 >= c0).astype(jnp.float32)
    ident = (r0 == c0).astype(jnp.float32)
    running = jnp.zeros((1, 128), jnp.float32)
    for c in range(a_tot // ch):
        mc = (icat[c * ch : (c + 1) * ch] == lane128).astype(jnp.float32)
        c_scr[c * ch : (c + 1) * ch, :] = (
            jnp.dot(tri, mc, preferred_element_type=jnp.float32, precision=_HI)
            + running
        )
        running = running + jnp.sum(mc, axis=0, keepdims=True)
    counts = running
    blocks = jnp.ceil(counts / BLK)
    r1 = lax.broadcasted_iota(jnp.int32, (128, 128), 0)
    c1 = lax.broadcasted_iota(jnp.int32, (128, 128), 1)
    upper = (r1 < c1).astype(jnp.float32)
    i128 = (r1 == c1).astype(jnp.float32)
    cumb_ex = jnp.dot(blocks, upper, preferred_element_type=jnp.float32,
                      precision=_HI)
    off_row = BLK * cumb_ex
    cumb_in = cumb_ex + blocks
    cumb_col = lax.dot_general(i128, cumb_in, (((1,), (1,)), ((), ())),
                               preferred_element_type=jnp.float32, precision=_HI)
    b_row = lax.broadcasted_iota(jnp.int32, (1, 128), 1).astype(jnp.float32)
    eblk = jnp.sum((cumb_col <= b_row).astype(jnp.float32), axis=0, keepdims=True)
    eblk_ref[...] = jnp.minimum(eblk, E - 1).astype(jnp.int32)
    for c in range(a_tot // ch):
        mc = (icat[c * ch : (c + 1) * ch] == lane128).astype(jnp.float32)
        cc = c_scr[c * ch : (c + 1) * ch, :]
        rank = jnp.sum(mc * (cc - 1.0), axis=-1)[:, None]
        offc = jnp.sum(mc * off_row, axis=-1)[:, None]
        slot_col = rank + offc
        row = lax.dot_general(slot_col, ident, (((0,), (0,)), ((), ())),
                              preferred_element_type=jnp.float32, precision=_HI)
        slot_ref[:, c * ch : (c + 1) * ch] = row.astype(jnp.int32)


def _dispatch(h, router_pad, interpret=False):
    m, d = h.shape
    return pl.pallas_call(
        _dispatch_kernel,
        grid=(1,),
        in_specs=[
            pl.BlockSpec((m, d), lambda i: (0, 0)),
            pl.BlockSpec((d, 128), lambda i: (0, 0)),
        ],
        out_specs=[
            pl.BlockSpec((1, 2 * S), lambda i: (0, 0)),
            pl.BlockSpec((m, 128), lambda i: (0, 0)),
            pl.BlockSpec((m, 128), lambda i: (0, 0)),
            pl.BlockSpec((1, 128), lambda i: (0, 0)),
        ],
        out_shape=[
            jax.ShapeDtypeStruct((1, 2 * S), jnp.int32),
            jax.ShapeDtypeStruct((m, 128), jnp.float32),
            jax.ShapeDtypeStruct((m, 128), jnp.float32),
            jax.ShapeDtypeStruct((1, 128), jnp.int32),
        ],
        scratch_shapes=[pltpu.VMEM((2 * S, 128), jnp.float32)],
        interpret=interpret,
    )(h, router_pad)


def _tokscatter_kernel(slot_ref, tok_ref):
    b = pl.program_id(0)
    s_row = slot_ref[...]
    p_col = b * 512 + lax.broadcasted_iota(jnp.int32, (512, 1), 0)
    oh = (s_row == p_col).astype(jnp.float32)
    ji = lax.broadcasted_iota(jnp.int32, (1, 2 * S), 1)
    trow = (ji % S).astype(jnp.float32)
    tok = lax.dot_general(trow, oh, (((1,), (1,)), ((), ())),
                          preferred_element_type=jnp.float32, precision=_HI)
    ones = jnp.ones((1, 2 * S), jnp.float32)
    cov = lax.dot_general(ones, oh, (((1,), (1,)), ((), ())),
                          preferred_element_type=jnp.float32, precision=_HI)
    pad_tok = (p_col.reshape(1, 512) % S).astype(jnp.float32)
    tok = tok + (1.0 - cov) * pad_tok
    tok_ref[...] = tok.reshape(1, 1, 512).astype(jnp.int32)


def _tokscatter(slot_row, interpret=False):
    return pl.pallas_call(
        _tokscatter_kernel,
        grid=(P // 512,),
        in_specs=[pl.BlockSpec((1, 2 * S), lambda b: (0, 0))],
        out_specs=pl.BlockSpec((1, 1, 512), lambda b: (b, 0, 0)),
        out_shape=jax.ShapeDtypeStruct((P // 512, 1, 512), jnp.int32),
        interpret=interpret,
    )(slot_row)


def _gemma_kernel(eblk_ref, xs_ref, w1_ref, w3_ref, hs_ref):
    x = xs_ref[...]
    a = _bdot(x, w1_ref[0])
    b = _bdot(x, w3_ref[0])
    hs_ref[...] = a * lax.logistic(a) * b


def _gemma(xs, ew1, ew3, eblk, interpret=False):
    grid_spec = pltpu.PrefetchScalarGridSpec(
        num_scalar_prefetch=1,
        grid=(NBLK,),
        in_specs=[
            pl.BlockSpec((BLK, D), lambda b, e: (b, 0)),
            pl.BlockSpec((1, D, FF), lambda b, e: (e[b], 0, 0)),
            pl.BlockSpec((1, D, FF), lambda b, e: (e[b], 0, 0)),
        ],
        out_specs=pl.BlockSpec((BLK, FF), lambda b, e: (b, 0)),
    )
    return pl.pallas_call(
        _gemma_kernel,
        grid_spec=grid_spec,
        out_shape=jax.ShapeDtypeStruct((P, FF), jnp.float32),
        interpret=interpret,
    )(eblk, xs, ew1, ew3)


def _gemmb_kernel(eblk_ref, hs_ref, w2_ref, ys_ref):
    ys_ref[...] = _bdot(hs_ref[...], w2_ref[0])


def _gemmb(hs, ew2, eblk, interpret=False):
    grid_spec = pltpu.PrefetchScalarGridSpec(
        num_scalar_prefetch=1,
        grid=(NBLK,),
        in_specs=[
            pl.BlockSpec((BLK, FF), lambda b, e: (b, 0)),
            pl.BlockSpec((1, FF, D), lambda b, e: (e[b], 0, 0)),
        ],
        out_specs=pl.BlockSpec((BLK, D), lambda b, e: (b, 0)),
    )
    return pl.pallas_call(
        _gemmb_kernel,
        grid_spec=grid_spec,
        out_shape=jax.ShapeDtypeStruct((P, D), jnp.float32),
        interpret=interpret,
    )(eblk, hs, ew2)


def _combine_kernel(x_ref, sh_ref, g0_ref, g1_ref, w1b_ref, w2b_ref, o_ref):
    o_ref[...] = (
        x_ref[...]
        + sh_ref[...]
        + w1b_ref[...][:, :1] * g0_ref[...]
        + w2b_ref[...][:, :1] * g1_ref[...]
    )


def _combine(x, sh, g0, g1, w1b, w2b, bm=256, interpret=False):
    m, d = x.shape
    return pl.pallas_call(
        _combine_kernel,
        grid=(m // bm,),
        in_specs=[
            pl.BlockSpec((bm, d), lambda i: (i, 0)),
            pl.BlockSpec((bm, d), lambda i: (i, 0)),
            pl.BlockSpec((bm, d), lambda i: (i, 0)),
            pl.BlockSpec((bm, d), lambda i: (i, 0)),
            pl.BlockSpec((bm, 128), lambda i: (i, 0)),
            pl.BlockSpec((bm, 128), lambda i: (i, 0)),
        ],
        out_specs=pl.BlockSpec((bm, d), lambda i: (i, 0)),
        out_shape=jax.ShapeDtypeStruct((m, d), jnp.float32),
        interpret=interpret,
    )(x, sh, g0, g1, w1b, w2b)


def _head_kernel(x_ref, fn_ref, cw_ref, cb_ref, o_ref):
    x = x_ref[...]
    r = lax.rsqrt(jnp.mean(x * x, axis=-1, keepdims=True) + 1e-6)
    normed = x * r * fn_ref[...]
    pooled = jnp.mean(normed, axis=0, keepdims=True)
    logits = jnp.dot(pooled, cw_ref[...], preferred_element_type=jnp.float32)
    logits = logits + cb_ref[...]
    lane = lax.broadcasted_iota(jnp.int32, logits.shape, 1)
    logits = jnp.where(lane < NCLS, logits, jnp.float32(-1e30))
    m = jnp.max(logits, axis=-1, keepdims=True)
    e = jnp.exp(logits - m)
    p = e / jnp.sum(e, axis=-1, keepdims=True)
    o_ref[...] = jnp.broadcast_to(p, o_ref.shape)


def _head(x, fn, cw_pad, cb_pad, interpret=False):
    m, d = x.shape
    return pl.pallas_call(
        _head_kernel,
        grid=(1,),
        in_specs=[
            pl.BlockSpec((m, d), lambda i: (0, 0)),
            pl.BlockSpec((1, d), lambda i: (0, 0)),
            pl.BlockSpec((d, 128), lambda i: (0, 0)),
            pl.BlockSpec((1, 128), lambda i: (0, 0)),
        ],
        out_specs=pl.BlockSpec((8, 128), lambda i: (0, 0)),
        out_shape=jax.ShapeDtypeStruct((8, 128), jnp.float32),
        interpret=interpret,
    )(x, fn.reshape(1, d), cw_pad, cb_pad)


def kernel(X, params):
    b, s = X.shape
    idx = X.reshape(-1).astype(jnp.int32)
    x = _sc_gather_rows(params["emb"], idx, b * s, 64)

    for lp in params["layers"]:
        h = _rmsnorm(x, lp["n1"])
        wqkv = jnp.concatenate([lp["wq"], lp["wk"], lp["wv"]], axis=1)
        qkv = _matmul(h, wqkv)
        q = qkv[:, :D].reshape(s, H, DK).transpose(1, 0, 2)
        k = qkv[:, D : 2 * D].reshape(s, H, DK).transpose(1, 0, 2)
        v = qkv[:, 2 * D :].reshape(s, H, DK).transpose(1, 0, 2)
        o = _attention(q, k, v)
        attn = o.transpose(1, 0, 2).reshape(s, D)
        x = _matmul_add(attn, lp["wo"], x)

        h2 = _rmsnorm(x, lp["n2"])
        router_pad = jnp.pad(lp["router"], ((0, 0), (0, 128 - E)))
        slot_row, w1b, w2b, eblk128 = _dispatch(h2, router_pad)
        slot = slot_row.reshape(-1)
        eblk = eblk128[0, :NBLK]
        tok = _tokscatter(slot_row).reshape(-1)
        xs = _sc_gather_rows(h2, tok, P, 40)
        hs = _gemma(xs, lp["ew1"], lp["ew3"], eblk)
        ys = _gemmb(hs, lp["ew2"], eblk)
        g = _sc_gather_rows(ys, slot, 2 * s, 32)
        shared = _swiglu(h2, lp["sw1"], lp["sw3"], lp["sw2"])
        x = _combine(x, shared, g[:s], g[s:], w1b, w2b)

    cw_pad = jnp.pad(params["cw"], ((0, 0), (0, 128 - NCLS)))
    cb_pad = jnp.pad(params["cb"], (0, 128 - NCLS)).reshape(1, 128)
    probs = _head(x, params["fn"], cw_pad, cb_pad)
    return probs[:1, :NCLS]


# speedup vs baseline: 1.0238x; 1.0238x over previous
import functools

import jax
import jax.numpy as jnp
import numpy as np
from jax import lax
from jax.experimental import pallas as pl
from jax.experimental.pallas import tpu as pltpu

D = 1024
H = 16
E = 8
FF = 2048
S = 2048
DK = D // H
NCLS = 10


def _bdot(a, b):
    return jnp.dot(a.astype(jnp.bfloat16), b.astype(jnp.bfloat16),
                   preferred_element_type=jnp.float32)


BLK = 128
P = 2 * S + E * BLK
NBLK = P // BLK


def _sc_gather_rows(table, idx, out_rows, ch):
    from jax.experimental.pallas import tpu_sc as plsc

    info = plsc.get_sparse_core_info()
    nc_, ns_ = info.num_cores, info.num_subcores
    nw = nc_ * ns_
    b_per_w = out_rows // nw
    nch = b_per_w // ch
    d = table.shape[1]
    mesh = plsc.VectorSubcoreMesh(core_axis_name="c", subcore_axis_name="s")

    nbuf = min(2, nch)

    @functools.partial(
        pl.kernel,
        mesh=mesh,
        out_type=jax.ShapeDtypeStruct((out_rows, d), jnp.float32),
        scratch_types=[
            pltpu.VMEM((b_per_w,), jnp.int32),
            pltpu.VMEM((nbuf, ch, d), jnp.float32),
            pltpu.SemaphoreType.DMA,
            pltpu.SemaphoreType.DMA,
        ],
    )
    def k(table_hbm, idx_hbm, out_hbm, idx_v, rows_v, sem0, sem1):
        wid = lax.axis_index("s") * nc_ + lax.axis_index("c")
        base = wid * b_per_w
        sems = [sem0, sem1]
        pltpu.sync_copy(idx_hbm.at[pl.ds(base, b_per_w)], idx_v)
        handles = {}
        for c in range(nbuf):
            handles[c] = pltpu.async_copy(
                table_hbm.at[idx_v.at[pl.ds(c * ch, ch)]],
                rows_v.at[c % nbuf], sems[c % nbuf])
        for c in range(nch):
            handles[c].wait()
            pltpu.sync_copy(rows_v.at[c % nbuf],
                            out_hbm.at[pl.ds(base + c * ch, ch)])
            nxt = c + nbuf
            if nxt < nch:
                handles[nxt] = pltpu.async_copy(
                    table_hbm.at[idx_v.at[pl.ds(nxt * ch, ch)]],
                    rows_v.at[nxt % nbuf], sems[nxt % nbuf])

    return k(table, idx)


def _rmsnorm_kernel(x_ref, w_ref, o_ref):
    x = x_ref[...]
    r = lax.rsqrt(jnp.mean(x * x, axis=-1, keepdims=True) + 1e-6)
    o_ref[...] = x * r * w_ref[...]


def _rmsnorm(x, w, bm=256, interpret=False):
    m, d = x.shape
    return pl.pallas_call(
        _rmsnorm_kernel,
        grid=(m // bm,),
        in_specs=[
            pl.BlockSpec((bm, d), lambda i: (i, 0)),
            pl.BlockSpec((1, d), lambda i: (0, 0)),
        ],
        out_specs=pl.BlockSpec((bm, d), lambda i: (i, 0)),
        out_shape=jax.ShapeDtypeStruct((m, d), jnp.float32),
        interpret=interpret,
    )(x, w.reshape(1, d))


def _mm_kernel(x_ref, w_ref, o_ref):
    o_ref[...] = _bdot(x_ref[...], w_ref[...])


def _matmul(x, w, bm=512, interpret=False):
    m, k = x.shape
    _, n = w.shape
    return pl.pallas_call(
        _mm_kernel,
        grid=(m // bm,),
        in_specs=[
            pl.BlockSpec((bm, k), lambda i: (i, 0)),
            pl.BlockSpec((k, n), lambda i: (0, 0)),
        ],
        out_specs=pl.BlockSpec((bm, n), lambda i: (i, 0)),
        out_shape=jax.ShapeDtypeStruct((m, n), jnp.float32),
        interpret=interpret,
    )(x, w)


def _mm_add_kernel(x_ref, w_ref, r_ref, o_ref):
    o_ref[...] = r_ref[...] + _bdot(x_ref[...], w_ref[...])


def _matmul_add(x, w, res, bm=512, interpret=False):
    m, k = x.shape
    _, n = w.shape
    return pl.pallas_call(
        _mm_add_kernel,
        grid=(m // bm,),
        in_specs=[
            pl.BlockSpec((bm, k), lambda i: (i, 0)),
            pl.BlockSpec((k, n), lambda i: (0, 0)),
            pl.BlockSpec((bm, n), lambda i: (i, 0)),
        ],
        out_specs=pl.BlockSpec((bm, n), lambda i: (i, 0)),
        out_shape=jax.ShapeDtypeStruct((m, n), jnp.float32),
        interpret=interpret,
    )(x, w, res)


def _attn_kernel(q_ref, k_ref, v_ref, o_ref):
    q = (q_ref[0] * (1.0 / np.sqrt(DK))).astype(jnp.bfloat16)
    k = k_ref[0].astype(jnp.bfloat16)
    v = v_ref[0].astype(jnp.bfloat16)
    s = jnp.dot(q, k.T, preferred_element_type=jnp.float32)
    m = jnp.max(s, axis=-1, keepdims=True)
    e = jnp.exp((s - m).astype(jnp.bfloat16))
    l = jnp.sum(e.astype(jnp.float32), axis=-1, keepdims=True)
    o = jnp.dot(e, v, preferred_element_type=jnp.float32)
    o_ref[0] = o / l


def _attention(q, k, v, bq=1024, interpret=False):
    return pl.pallas_call(
        _attn_kernel,
        grid=(H, S // bq),
        in_specs=[
            pl.BlockSpec((1, bq, DK), lambda h, i: (h, i, 0)),
            pl.BlockSpec((1, S, DK), lambda h, i: (h, 0, 0)),
            pl.BlockSpec((1, S, DK), lambda h, i: (h, 0, 0)),
        ],
        out_specs=pl.BlockSpec((1, bq, DK), lambda h, i: (h, i, 0)),
        out_shape=jax.ShapeDtypeStruct((H, S, DK), jnp.float32),
        interpret=interpret,
    )(q, k, v)


def _router_kernel(x_ref, w_ref, o_ref):
    logits = jnp.dot(x_ref[...], w_ref[...], preferred_element_type=jnp.float32)
    lane = lax.broadcasted_iota(jnp.int32, logits.shape, 1)
    neg = jnp.float32(-1e30)
    logits = jnp.where(lane < E, logits, neg)
    m = jnp.max(logits, axis=-1, keepdims=True)
    ex = jnp.where(lane < E, jnp.exp(logits - m), 0.0)
    probs = ex / jnp.sum(ex, axis=-1, keepdims=True)
    i1 = jnp.argmax(probs, axis=-1)
    m1 = jnp.max(probs, axis=-1)
    p2 = jnp.where(lane == i1[:, None], neg, jnp.where(lane < E, probs, neg))
    i2 = jnp.argmax(p2, axis=-1)
    m2 = jnp.max(p2, axis=-1)
    e1 = jnp.exp(m1)
    e2 = jnp.exp(m2)
    w1 = e1 / (e1 + e2)
    w2 = e2 / (e1 + e2)
    tw = w1[:, None] * (lane == i1[:, None]) + w2[:, None] * (lane == i2[:, None])
    o_ref[...] = tw.astype(jnp.float32)


def _router(x, router_w, bm=256, interpret=False):
    m, d = x.shape
    return pl.pallas_call(
        _router_kernel,
        grid=(m // bm,),
        in_specs=[
            pl.BlockSpec((bm, d), lambda i: (i, 0)),
            pl.BlockSpec((d, 128), lambda i: (0, 0)),
        ],
        out_specs=pl.BlockSpec((bm, 128), lambda i: (i, 0)),
        out_shape=jax.ShapeDtypeStruct((m, 128), jnp.float32),
        interpret=interpret,
    )(x, router_w)


def _swiglu_kernel(x_ref, w1_ref, w3_ref, w2_ref, o_ref):
    x = x_ref[...]
    a = _bdot(x, w1_ref[...])
    b = _bdot(x, w3_ref[...])
    h = (a * jax.lax.logistic(a)) * b
    o_ref[...] = _bdot(h, w2_ref[...])


def _swiglu(x, w1, w3, w2, bm=512, interpret=False):
    m, d = x.shape
    ff = w1.shape[1]
    return pl.pallas_call(
        _swiglu_kernel,
        grid=(m // bm,),
        in_specs=[
            pl.BlockSpec((bm, d), lambda i: (i, 0)),
            pl.BlockSpec((d, ff), lambda i: (0, 0)),
            pl.BlockSpec((d, ff), lambda i: (0, 0)),
            pl.BlockSpec((ff, d), lambda i: (0, 0)),
        ],
        out_specs=pl.BlockSpec((bm, d), lambda i: (i, 0)),
        out_shape=jax.ShapeDtypeStruct((m, d), jnp.float32),
        interpret=interpret,
    )(x, w1, w3, w2)


_HI = lax.Precision.HIGHEST


def _dispatch_kernel(h_ref, r_ref, slot_ref, w1b_ref, w2b_ref, eblk_ref, c_scr):
    h = h_ref[...]
    logits = jnp.dot(h, r_ref[...], preferred_element_type=jnp.float32)
    lane = lax.broadcasted_iota(jnp.int32, logits.shape, 1)
    neg = jnp.float32(-1e30)
    logits = jnp.where(lane < E, logits, neg)
    m = jnp.max(logits, axis=-1, keepdims=True)
    ex = jnp.where(lane < E, jnp.exp(logits - m), 0.0)
    probs = ex / jnp.sum(ex, axis=-1, keepdims=True)
    pm = jnp.where(lane < E, probs, neg)
    i1 = jnp.argmax(pm, axis=-1)
    m1 = jnp.max(pm, axis=-1)
    p2 = jnp.where(lane == i1[:, None], neg, pm)
    i2 = jnp.argmax(p2, axis=-1)
    m2 = jnp.max(p2, axis=-1)
    e1 = jnp.exp(m1)
    e2 = jnp.exp(m2)
    wa = e1 / (e1 + e2)
    wb = e2 / (e1 + e2)
    w1b_ref[...] = jnp.broadcast_to(wa[:, None], w1b_ref.shape)
    w2b_ref[...] = jnp.broadcast_to(wb[:, None], w2b_ref.shape)

    icat = jnp.concatenate([i1[:, None], i2[:, None]], axis=0)
    a_tot = 2 * S
    ch = 512
    lane128 = lax.broadcasted_iota(jnp.int32, (1, 128), 1)
    r0 = lax.broadcasted_iota(jnp.int32, (ch, ch), 0)
    c0 = lax.broadcasted_iota(jnp.int32, (ch, ch), 1)
    tri = (r0 >= c0).astype(jnp.float32)
    ident = (r0 == c0).astype(jnp.float32)
    running = jnp.zeros((1, 128), jnp.float32)
    for c in range(a_tot // ch):
        mc = (icat[c * ch : (c + 1) * ch] == lane128).astype(jnp.float32)
        c_scr[c * ch : (c + 1) * ch, :] = (
            jnp.dot(tri, mc, preferred_element_type=jnp.float32, precision=_HI)
            + running
        )
        running = running + jnp.sum(mc, axis=0, keepdims=True)
    counts = running
    blocks = jnp.ceil(counts / BLK)
    r1 = lax.broadcasted_iota(jnp.int32, (128, 128), 0)
    c1 = lax.broadcasted_iota(jnp.int32, (128, 128), 1)
    upper = (r1 < c1).astype(jnp.float32)
    i128 = (r1 == c1).astype(jnp.float32)
    cumb_ex = jnp.dot(blocks, upper, preferred_element_type=jnp.float32,
                      precision=_HI)
    off_row = BLK * cumb_ex
    cumb_in = cumb_ex + blocks
    cumb_col = lax.dot_general(i128, cumb_in, (((1,), (1,)), ((), ())),
                               preferred_element_type=jnp.float32, precision=_HI)
    b_row = lax.broadcasted_iota(jnp.int32, (1, 128), 1).astype(jnp.float32)
    eblk = jnp.sum((cumb_col <= b_row).astype(jnp.float32), axis=0, keepdims=True)
    eblk_ref[...] = jnp.minimum(eblk, E - 1).astype(jnp.int32)
    for c in range(a_tot // ch):
        mc = (icat[c * ch : (c + 1) * ch] == lane128).astype(jnp.float32)
        cc = c_scr[c * ch : (c + 1) * ch, :]
        rank = jnp.sum(mc * (cc - 1.0), axis=-1)[:, None]
        offc = jnp.sum(mc * off_row, axis=-1)[:, None]
        slot_col = rank + offc
        row = lax.dot_general(slot_col, ident, (((0,), (0,)), ((), ())),
                              preferred_element_type=jnp.float32, precision=_HI)
        slot_ref[:, c * ch : (c + 1) * ch] = row.astype(jnp.int32)


def _dispatch(h, router_pad, interpret=False):
    m, d = h.shape
    return pl.pallas_call(
        _dispatch_kernel,
        grid=(1,),
        in_specs=[
            pl.BlockSpec((m, d), lambda i: (0, 0)),
            pl.BlockSpec((d, 128), lambda i: (0, 0)),
        ],
        out_specs=[
            pl.BlockSpec((1, 2 * S), lambda i: (0, 0)),
            pl.BlockSpec((m, 128), lambda i: (0, 0)),
            pl.BlockSpec((m, 128), lambda i: (0, 0)),
            pl.BlockSpec((1, 128), lambda i: (0, 0)),
        ],
        out_shape=[
            jax.ShapeDtypeStruct((1, 2 * S), jnp.int32),
            jax.ShapeDtypeStruct((m, 128), jnp.float32),
            jax.ShapeDtypeStruct((m, 128), jnp.float32),
            jax.ShapeDtypeStruct((1, 128), jnp.int32),
        ],
        scratch_shapes=[pltpu.VMEM((2 * S, 128), jnp.float32)],
        interpret=interpret,
    )(h, router_pad)


def _tokscatter_kernel(slot_ref, tok_ref):
    b = pl.program_id(0)
    s_row = slot_ref[...]
    p_col = b * 512 + lax.broadcasted_iota(jnp.int32, (512, 1), 0)
    oh = (s_row == p_col).astype(jnp.float32)
    ji = lax.broadcasted_iota(jnp.int32, (1, 2 * S), 1)
    trow = (ji % S).astype(jnp.float32)
    tok = lax.dot_general(trow, oh, (((1,), (1,)), ((), ())),
                          preferred_element_type=jnp.float32, precision=_HI)
    ones = jnp.ones((1, 2 * S), jnp.float32)
    cov = lax.dot_general(ones, oh, (((1,), (1,)), ((), ())),
                          preferred_element_type=jnp.float32, precision=_HI)
    pad_tok = (p_col.reshape(1, 512) % S).astype(jnp.float32)
    tok = tok + (1.0 - cov) * pad_tok
    tok_ref[...] = tok.reshape(1, 1, 512).astype(jnp.int32)


def _tokscatter(slot_row, interpret=False):
    return pl.pallas_call(
        _tokscatter_kernel,
        grid=(P // 512,),
        in_specs=[pl.BlockSpec((1, 2 * S), lambda b: (0, 0))],
        out_specs=pl.BlockSpec((1, 1, 512), lambda b: (b, 0, 0)),
        out_shape=jax.ShapeDtypeStruct((P // 512, 1, 512), jnp.int32),
        interpret=interpret,
    )(slot_row)


def _gemma_kernel(eblk_ref, xs_ref, w1_ref, w3_ref, hs_ref):
    x = xs_ref[...]
    a = _bdot(x, w1_ref[0])
    b = _bdot(x, w3_ref[0])
    hs_ref[...] = a * lax.logistic(a) * b


def _gemma(xs, ew1, ew3, eblk, interpret=False):
    grid_spec = pltpu.PrefetchScalarGridSpec(
        num_scalar_prefetch=1,
        grid=(NBLK,),
        in_specs=[
            pl.BlockSpec((BLK, D), lambda b, e: (b, 0)),
            pl.BlockSpec((1, D, FF), lambda b, e: (e[b], 0, 0)),
            pl.BlockSpec((1, D, FF), lambda b, e: (e[b], 0, 0)),
        ],
        out_specs=pl.BlockSpec((BLK, FF), lambda b, e: (b, 0)),
    )
    return pl.pallas_call(
        _gemma_kernel,
        grid_spec=grid_spec,
        out_shape=jax.ShapeDtypeStruct((P, FF), jnp.float32),
        interpret=interpret,
    )(eblk, xs, ew1, ew3)


def _gemmb_kernel(eblk_ref, hs_ref, w2_ref, ys_ref):
    ys_ref[...] = _bdot(hs_ref[...], w2_ref[0])


def _gemmb(hs, ew2, eblk, interpret=False):
    grid_spec = pltpu.PrefetchScalarGridSpec(
        num_scalar_prefetch=1,
        grid=(NBLK,),
        in_specs=[
            pl.BlockSpec((BLK, FF), lambda b, e: (b, 0)),
            pl.BlockSpec((1, FF, D), lambda b, e: (e[b], 0, 0)),
        ],
        out_specs=pl.BlockSpec((BLK, D), lambda b, e: (b, 0)),
    )
    return pl.pallas_call(
        _gemmb_kernel,
        grid_spec=grid_spec,
        out_shape=jax.ShapeDtypeStruct((P, D), jnp.float32),
        interpret=interpret,
    )(eblk, hs, ew2)


def _combine_kernel(x_ref, sh_ref, g0_ref, g1_ref, w1b_ref, w2b_ref, o_ref):
    o_ref[...] = (
        x_ref[...]
        + sh_ref[...]
        + w1b_ref[...][:, :1] * g0_ref[...]
        + w2b_ref[...][:, :1] * g1_ref[...]
    )


def _combine(x, sh, g0, g1, w1b, w2b, bm=256, interpret=False):
    m, d = x.shape
    return pl.pallas_call(
        _combine_kernel,
        grid=(m // bm,),
        in_specs=[
            pl.BlockSpec((bm, d), lambda i: (i, 0)),
            pl.BlockSpec((bm, d), lambda i: (i, 0)),
            pl.BlockSpec((bm, d), lambda i: (i, 0)),
            pl.BlockSpec((bm, d), lambda i: (i, 0)),
            pl.BlockSpec((bm, 128), lambda i: (i, 0)),
            pl.BlockSpec((bm, 128), lambda i: (i, 0)),
        ],
        out_specs=pl.BlockSpec((bm, d), lambda i: (i, 0)),
        out_shape=jax.ShapeDtypeStruct((m, d), jnp.float32),
        interpret=interpret,
    )(x, sh, g0, g1, w1b, w2b)


def _head_kernel(x_ref, fn_ref, cw_ref, cb_ref, o_ref):
    x = x_ref[...]
    r = lax.rsqrt(jnp.mean(x * x, axis=-1, keepdims=True) + 1e-6)
    normed = x * r * fn_ref[...]
    pooled = jnp.mean(normed, axis=0, keepdims=True)
    logits = jnp.dot(pooled, cw_ref[...], preferred_element_type=jnp.float32)
    logits = logits + cb_ref[...]
    lane = lax.broadcasted_iota(jnp.int32, logits.shape, 1)
    logits = jnp.where(lane < NCLS, logits, jnp.float32(-1e30))
    m = jnp.max(logits, axis=-1, keepdims=True)
    e = jnp.exp(logits - m)
    p = e / jnp.sum(e, axis=-1, keepdims=True)
    o_ref[...] = jnp.broadcast_to(p, o_ref.shape)


def _head(x, fn, cw_pad, cb_pad, interpret=False):
    m, d = x.shape
    return pl.pallas_call(
        _head_kernel,
        grid=(1,),
        in_specs=[
            pl.BlockSpec((m, d), lambda i: (0, 0)),
            pl.BlockSpec((1, d), lambda i: (0, 0)),
            pl.BlockSpec((d, 128), lambda i: (0, 0)),
            pl.BlockSpec((1, 128), lambda i: (0, 0)),
        ],
        out_specs=pl.BlockSpec((8, 128), lambda i: (0, 0)),
        out_shape=jax.ShapeDtypeStruct((8, 128), jnp.float32),
        interpret=interpret,
    )(x, fn.reshape(1, d), cw_pad, cb_pad)


def kernel(X, params):
    b, s = X.shape
    idx = X.reshape(-1).astype(jnp.int32)
    x = _sc_gather_rows(params["emb"], idx, b * s, 64)

    for lp in params["layers"]:
        h = _rmsnorm(x, lp["n1"])
        wqkv = jnp.concatenate([lp["wq"], lp["wk"], lp["wv"]], axis=1)
        qkv = _matmul(h, wqkv)
        q = qkv[:, :D].reshape(s, H, DK).transpose(1, 0, 2)
        k = qkv[:, D : 2 * D].reshape(s, H, DK).transpose(1, 0, 2)
        v = qkv[:, 2 * D :].reshape(s, H, DK).transpose(1, 0, 2)
        o = _attention(q, k, v)
        attn = o.transpose(1, 0, 2).reshape(s, D)
        x = _matmul_add(attn, lp["wo"], x)

        h2 = _rmsnorm(x, lp["n2"])
        router_pad = jnp.pad(lp["router"], ((0, 0), (0, 128 - E)))
        slot_row, w1b, w2b, eblk128 = _dispatch(h2, router_pad)
        slot = slot_row.reshape(-1)
        eblk = eblk128[0, :NBLK]
        tok = _tokscatter(slot_row).reshape(-1)
        xs = _sc_gather_rows(h2, tok, P, 40)
        hs = _gemma(xs, lp["ew1"], lp["ew3"], eblk)
        ys = _gemmb(hs, lp["ew2"], eblk)
        g = _sc_gather_rows(ys, slot, 2 * s, 32)
        shared = _swiglu(h2, lp["sw1"], lp["sw3"], lp["sw2"])
        x = _combine(x, shared, g[:s], g[s:], w1b, w2b)

    cw_pad = jnp.pad(params["cw"], ((0, 0), (0, 128 - NCLS)))
    cb_pad = jnp.pad(params["cb"], (0, 128 - NCLS)).reshape(1, 128)
    probs = _head(x, params["fn"], cw_pad, cb_pad)
    return probs[:1, :NCLS]
